# baseline (speedup 1.0000x reference)
import functools

import jax
import jax.numpy as jnp
from jax import lax
from jax.experimental import pallas as pl
from jax.experimental.pallas import tpu as pltpu
from jax.experimental.pallas import tpu_sc as plsc

D = 128
N = 2 * 512 * 512
NC, NS = 2, 16
NW = NC * NS
N_SC = 18 * 16384
N_TC = N - N_SC
ROWS_PER_W = max(N_SC // NW, 512)
CHUNK = 256
PAIRS = ROWS_PER_W // (2 * CHUNK)
GROUP = 16
B_TC = 4096


def _compute_chunk(buf, outv, ms, is_, out_base):
    lane = jnp.arange(16, dtype=jnp.int32)

    @plsc.parallel_loop(0, CHUNK // GROUP, unroll=2, carry=jnp.int32(0))
    def group_body(g, c):
        gv = jnp.zeros((16,), jnp.int32) + g
        base = g * GROUP
        for rr in range(GROUP):
            r = base + rr
            m = buf[r, pl.ds(0, 16)]
            idx = lane
            for k in range(1, D // 16):
                v = buf[r, pl.ds(k * 16, 16)]
                gt = v > m
                m = jnp.where(gt, v, m)
                idx = jnp.where(gt, lane + (16 * k), idx)
            addr = rr * 16 + ((lane + rr) & 15)
            plsc.store_scatter(ms, [gv, addr], m)
            plsc.store_scatter(is_, [gv, addr], idx)
        mv = None
        for j in range(GROUP):
            addr = lane * 16 + ((lane + j) & 15)
            tm = plsc.load_gather(ms, [gv, addr])
            ti = plsc.load_gather(is_, [gv, addr])
            if mv is None:
                mv, iv = tm, ti
            else:
                gt = tm > mv
                eq = tm == mv
                lt = ti < iv
                take = gt | (eq & lt)
                mv = jnp.where(take, tm, mv)
                iv = jnp.where(take, ti, iv)
        outv[pl.ds(out_base + base, GROUP)] = iv
        return c


def _sc_argmax_body(x_hbm, out_hbm, buf_a, buf_b, outv, ms, is_, sem_a, sem_b):
    wid = lax.axis_index("s") * NC + lax.axis_index("c")
    row0 = wid * ROWS_PER_W

    def start(base, buf, sem):
        pltpu.make_async_copy(x_hbm.at[pl.ds(base, CHUNK)], buf, sem).start()

    def wait(base, buf, sem):
        pltpu.make_async_copy(x_hbm.at[pl.ds(base, CHUNK)], buf, sem).wait()

    start(row0, buf_a, sem_a)

    def ring_body(g, carry):
        base_a = row0 + (2 * g) * CHUNK
        base_b = base_a + CHUNK
        start(base_b, buf_b, sem_b)
        wait(base_a, buf_a, sem_a)
        _compute_chunk(buf_a, outv, ms, is_, (2 * g) * CHUNK)

        @pl.when(g < PAIRS - 1)
        def _():
            start(base_b + CHUNK, buf_a, sem_a)

        wait(base_b, buf_b, sem_b)
        _compute_chunk(buf_b, outv, ms, is_, (2 * g + 1) * CHUNK)
        return carry

    lax.fori_loop(0, PAIRS, ring_body, 0, unroll=False)
    pltpu.sync_copy(outv, out_hbm.at[pl.ds(row0, ROWS_PER_W)])


def _tc_body(x_ref, o_ref):
    x = x_ref[...]
    o_ref[...] = jnp.argmax(x, axis=1).astype(jnp.int32)


def _sc_call(x):
    mesh = plsc.VectorSubcoreMesh(
        core_axis_name="c", subcore_axis_name="s",
        num_cores=NC, num_subcores=NS)
    return pl.kernel(
        _sc_argmax_body,
        out_type=jax.ShapeDtypeStruct((N_SC,), jnp.int32),
        mesh=mesh,
        compiler_params=pltpu.CompilerParams(needs_layout_passes=False),
        scratch_types=[
            pltpu.VMEM((CHUNK, D), jnp.float32),
            pltpu.VMEM((CHUNK, D), jnp.float32),
            pltpu.VMEM((ROWS_PER_W,), jnp.int32),
            pltpu.VMEM((CHUNK // GROUP, GROUP * GROUP), jnp.float32),
            pltpu.VMEM((CHUNK // GROUP, GROUP * GROUP), jnp.int32),
            pltpu.SemaphoreType.DMA,
            pltpu.SemaphoreType.DMA,
        ],
    )(x)


def _tc_call(x):
    blk0 = N_SC // B_TC
    return pl.pallas_call(
        _tc_body,
        grid=(N_TC // B_TC,),
        in_specs=[pl.BlockSpec((B_TC, D), lambda i: (i + blk0, 0))],
        out_specs=pl.BlockSpec((B_TC,), lambda i: (i,)),
        out_shape=jax.ShapeDtypeStruct((N_TC,), jnp.int32),
    )(x)


@jax.jit
def kernel(beliefs):
    x = beliefs.reshape(N, D)
    parts = []
    if N_SC:
        parts.append(_sc_call(x))
    if N_TC:
        parts.append(_tc_call(x))
    out = jnp.concatenate(parts) if len(parts) > 1 else parts[0]
    return out.reshape(2, 512, 512, 1)


# speedup vs baseline: 1.0492x; 1.0492x over previous
import functools

import jax
import jax.numpy as jnp
from jax import lax
from jax.experimental import pallas as pl
from jax.experimental.pallas import tpu as pltpu
from jax.experimental.pallas import tpu_sc as plsc

D = 128
N = 2 * 512 * 512
NC, NS = 2, 16
NW = NC * NS
N_SC = 17 * 16384
N_TC = N - N_SC
ROWS_PER_W = max(N_SC // NW, 512)
CHUNK = 256
PAIRS = ROWS_PER_W // (2 * CHUNK)
GROUP = 16
B_TC = 8192


def _compute_chunk(buf, outv, ms, is_, out_base):
    lane = jnp.arange(16, dtype=jnp.int32)

    @plsc.parallel_loop(0, CHUNK // GROUP, unroll=2, carry=jnp.int32(0))
    def group_body(g, c):
        gv = jnp.zeros((16,), jnp.int32) + g
        base = g * GROUP
        for rr in range(GROUP):
            r = base + rr
            m = buf[r, pl.ds(0, 16)]
            idx = lane
            for k in range(1, D // 16):
                v = buf[r, pl.ds(k * 16, 16)]
                gt = v > m
                m = jnp.where(gt, v, m)
                idx = jnp.where(gt, lane + (16 * k), idx)
            addr = rr * 16 + ((lane + rr) & 15)
            plsc.store_scatter(ms, [gv, addr], m)
            plsc.store_scatter(is_, [gv, addr], idx)
        mv = None
        for j in range(GROUP):
            addr = lane * 16 + ((lane + j) & 15)
            tm = plsc.load_gather(ms, [gv, addr])
            ti = plsc.load_gather(is_, [gv, addr])
            if mv is None:
                mv, iv = tm, ti
            else:
                gt = tm > mv
                eq = tm == mv
                lt = ti < iv
                take = gt | (eq & lt)
                mv = jnp.where(take, tm, mv)
                iv = jnp.where(take, ti, iv)
        outv[pl.ds(out_base + base, GROUP)] = iv
        return c


def _sc_argmax_body(x_hbm, out_hbm, buf_a, buf_b, outv, ms, is_, sem_a, sem_b):
    wid = lax.axis_index("s") * NC + lax.axis_index("c")
    row0 = wid * ROWS_PER_W

    def start(base, buf, sem):
        pltpu.make_async_copy(x_hbm.at[pl.ds(base, CHUNK)], buf, sem).start()

    def wait(base, buf, sem):
        pltpu.make_async_copy(x_hbm.at[pl.ds(base, CHUNK)], buf, sem).wait()

    start(row0, buf_a, sem_a)

    def ring_body(g, carry):
        base_a = row0 + (2 * g) * CHUNK
        base_b = base_a + CHUNK
        start(base_b, buf_b, sem_b)
        wait(base_a, buf_a, sem_a)
        _compute_chunk(buf_a, outv, ms, is_, (2 * g) * CHUNK)

        @pl.when(g < PAIRS - 1)
        def _():
            start(base_b + CHUNK, buf_a, sem_a)

        wait(base_b, buf_b, sem_b)
        _compute_chunk(buf_b, outv, ms, is_, (2 * g + 1) * CHUNK)
        return carry

    lax.fori_loop(0, PAIRS, ring_body, 0, unroll=False)
    pltpu.sync_copy(outv, out_hbm.at[pl.ds(row0, ROWS_PER_W)])


def _tc_body(x_ref, o_ref):
    x = x_ref[...]
    o_ref[...] = jnp.argmax(x, axis=1).astype(jnp.int32)


def _sc_call(x):
    mesh = plsc.VectorSubcoreMesh(
        core_axis_name="c", subcore_axis_name="s",
        num_cores=NC, num_subcores=NS)
    return pl.kernel(
        _sc_argmax_body,
        out_type=jax.ShapeDtypeStruct((N_SC,), jnp.int32),
        mesh=mesh,
        compiler_params=pltpu.CompilerParams(needs_layout_passes=False),
        scratch_types=[
            pltpu.VMEM((CHUNK, D), jnp.float32),
            pltpu.VMEM((CHUNK, D), jnp.float32),
            pltpu.VMEM((ROWS_PER_W,), jnp.int32),
            pltpu.VMEM((CHUNK // GROUP, GROUP * GROUP), jnp.float32),
            pltpu.VMEM((CHUNK // GROUP, GROUP * GROUP), jnp.int32),
            pltpu.SemaphoreType.DMA,
            pltpu.SemaphoreType.DMA,
        ],
    )(x)


def _tc_call(x):
    blk0 = N_SC // B_TC
    return pl.pallas_call(
        _tc_body,
        grid=(N_TC // B_TC,),
        in_specs=[pl.BlockSpec((B_TC, D), lambda i: (i + blk0, 0))],
        out_specs=pl.BlockSpec((B_TC,), lambda i: (i,)),
        out_shape=jax.ShapeDtypeStruct((N_TC,), jnp.int32),
    )(x)


@jax.jit
def kernel(beliefs):
    x = beliefs.reshape(N, D)
    parts = []
    if N_SC:
        parts.append(_sc_call(x))
    if N_TC:
        parts.append(_tc_call(x))
    out = jnp.concatenate(parts) if len(parts) > 1 else parts[0]
    return out.reshape(2, 512, 512, 1)


# speedup vs baseline: 1.1023x; 1.0506x over previous
import functools

import jax
import jax.numpy as jnp
from jax import lax
from jax.experimental import pallas as pl
from jax.experimental.pallas import tpu as pltpu
from jax.experimental.pallas import tpu_sc as plsc

D = 128
N = 2 * 512 * 512
NC, NS = 2, 16
NW = NC * NS
N_SC = 18 * 16384
N_TC = N - N_SC
ROWS_PER_W = max(N_SC // NW, 512)
CHUNK = 256
PAIRS = ROWS_PER_W // (2 * CHUNK)
GROUP = 16
B_TC = 8192


def _compute_chunk(buf, outv, ms, is_, out_base):
    lane = jnp.arange(16, dtype=jnp.int32)

    @plsc.parallel_loop(0, CHUNK // GROUP, unroll=2, carry=jnp.int32(0))
    def group_body(g, c):
        gv = jnp.zeros((16,), jnp.int32) + g
        base = g * GROUP
        for rr in range(GROUP):
            r = base + rr
            m = buf[r, pl.ds(0, 16)]
            idx = lane
            for k in range(1, D // 16):
                v = buf[r, pl.ds(k * 16, 16)]
                gt = v > m
                m = jnp.where(gt, v, m)
                idx = jnp.where(gt, lane + (16 * k), idx)
            addr = rr * 16 + ((lane + rr) & 15)
            plsc.store_scatter(ms, [gv, addr], m)
            plsc.store_scatter(is_, [gv, addr], idx)
        mv = None
        for j in range(GROUP):
            addr = lane * 16 + ((lane + j) & 15)
            tm = plsc.load_gather(ms, [gv, addr])
            ti = plsc.load_gather(is_, [gv, addr])
            if mv is None:
                mv, iv = tm, ti
            else:
                gt = tm > mv
                eq = tm == mv
                lt = ti < iv
                take = gt | (eq & lt)
                mv = jnp.where(take, tm, mv)
                iv = jnp.where(take, ti, iv)
        outv[pl.ds(out_base + base, GROUP)] = iv
        return c


def _sc_argmax_body(x_hbm, out_hbm, buf_a, buf_b, outv, ms, is_, sem_a, sem_b):
    wid = lax.axis_index("s") * NC + lax.axis_index("c")
    row0 = wid * ROWS_PER_W

    def start(base, buf, sem):
        pltpu.make_async_copy(x_hbm.at[pl.ds(base, CHUNK)], buf, sem).start()

    def wait(base, buf, sem):
        pltpu.make_async_copy(x_hbm.at[pl.ds(base, CHUNK)], buf, sem).wait()

    start(row0, buf_a, sem_a)

    def ring_body(g, carry):
        base_a = row0 + (2 * g) * CHUNK
        base_b = base_a + CHUNK
        start(base_b, buf_b, sem_b)
        wait(base_a, buf_a, sem_a)
        _compute_chunk(buf_a, outv, ms, is_, (2 * g) * CHUNK)

        @pl.when(g < PAIRS - 1)
        def _():
            start(base_b + CHUNK, buf_a, sem_a)

        wait(base_b, buf_b, sem_b)
        _compute_chunk(buf_b, outv, ms, is_, (2 * g + 1) * CHUNK)
        return carry

    lax.fori_loop(0, PAIRS, ring_body, 0, unroll=False)
    pltpu.sync_copy(outv, out_hbm.at[pl.ds(row0, ROWS_PER_W)])


def _tc_body(x_ref, o_ref):
    x = x_ref[...]
    o_ref[...] = jnp.argmax(x, axis=1).astype(jnp.int32)


def _sc_call(x):
    mesh = plsc.VectorSubcoreMesh(
        core_axis_name="c", subcore_axis_name="s",
        num_cores=NC, num_subcores=NS)
    return pl.kernel(
        _sc_argmax_body,
        out_type=jax.ShapeDtypeStruct((N_SC,), jnp.int32),
        mesh=mesh,
        compiler_params=pltpu.CompilerParams(needs_layout_passes=False),
        scratch_types=[
            pltpu.VMEM((CHUNK, D), jnp.float32),
            pltpu.VMEM((CHUNK, D), jnp.float32),
            pltpu.VMEM((ROWS_PER_W,), jnp.int32),
            pltpu.VMEM((CHUNK // GROUP, GROUP * GROUP), jnp.float32),
            pltpu.VMEM((CHUNK // GROUP, GROUP * GROUP), jnp.int32),
            pltpu.SemaphoreType.DMA,
            pltpu.SemaphoreType.DMA,
        ],
    )(x)


def _tc_call(x):
    blk0 = N_SC // B_TC
    return pl.pallas_call(
        _tc_body,
        grid=(N_TC // B_TC,),
        in_specs=[pl.BlockSpec((B_TC, D), lambda i: (i + blk0, 0))],
        out_specs=pl.BlockSpec((B_TC,), lambda i: (i,)),
        out_shape=jax.ShapeDtypeStruct((N_TC,), jnp.int32),
    )(x)


@jax.jit
def kernel(beliefs):
    x = beliefs.reshape(N, D)
    parts = []
    if N_SC:
        parts.append(_sc_call(x))
    if N_TC:
        parts.append(_tc_call(x))
    out = jnp.concatenate(parts) if len(parts) > 1 else parts[0]
    return out.reshape(2, 512, 512, 1)
